# Initial kernel scaffold; baseline (speedup 1.0000x reference)
import functools

import jax
import jax.numpy as jnp
from jax import lax
from jax.experimental import pallas as pl
from jax.experimental.pallas import tpu as pltpu
from jax.experimental.pallas import tpu_sc as plsc

N = 10000
NP = 10240
E = 320000
D = 128
H = 16

NC = 2
NS = 16
NW = NC * NS

EB = 128
ETB = 79
ET = EB * ETB
EPAD = ET * NW
NBLK = EPAD // EB

ROWS_PER_TILE = NP // NS
WCHUNKS = ROWS_PER_TILE // EB

_mesh = plsc.VectorSubcoreMesh(core_axis_name="c", subcore_axis_name="s")


def _zero_rows(buf):
    z = jnp.zeros((H,), jnp.float32)
    for i in range(EB):
        buf[i] = z


def _fill_ones(buf):
    o = jnp.ones((H,), jnp.float32)
    for i in range(EB):
        buf[i] = o


def _writeout(acc, tmp, out_hbm, c, s):
    for k in range(WCHUNKS):
        off = s * ROWS_PER_TILE + k * EB
        pltpu.sync_copy(acc.at[pl.ds(off, EB)], tmp)
        pltpu.sync_copy(tmp, out_hbm.at[pl.ds(c * NP + off, EB)])


def _deg_body(dstb_hbm, out_hbm, dst2d, ones2d, tmp, acc):
    c = lax.axis_index("c")
    s = lax.axis_index("s")
    w = c * NS + s
    pltpu.sync_copy(dstb_hbm.at[pl.ds(w * ETB, ETB)], dst2d)
    _fill_ones(ones2d)
    _zero_rows(tmp)
    for k in range(WCHUNKS):
        pltpu.sync_copy(tmp, acc.at[pl.ds(s * ROWS_PER_TILE + k * EB, EB)])
    plsc.subcore_barrier()

    def blk(j, carry):
        pltpu.sync_copy(ones2d, acc.at[dst2d.at[j]], add=True)
        return carry

    lax.fori_loop(0, ETB, blk, 0)
    plsc.subcore_barrier()
    _writeout(acc, tmp, out_hbm, c, s)


_deg_kernel = pl.kernel(
    _deg_body,
    out_type=jax.ShapeDtypeStruct((NC * NP, H), jnp.float32),
    mesh=_mesh,
    scratch_types=[
        pltpu.VMEM((ETB, EB), jnp.int32),
        pltpu.VMEM((EB, H), jnp.float32),
        pltpu.VMEM((EB, H), jnp.float32),
        pltpu.VMEM_SHARED((NP, H), jnp.float32),
    ],
)


def _edge_body(hs_hbm, srcb_hbm, dstb_hbm, out_hbm, src2d, dst2d, rows, tmp,
               acc, sem):
    c = lax.axis_index("c")
    s = lax.axis_index("s")
    w = c * NS + s
    pltpu.sync_copy(srcb_hbm.at[pl.ds(w * ETB, ETB)], src2d)
    pltpu.sync_copy(dstb_hbm.at[pl.ds(w * ETB, ETB)], dst2d)
    _zero_rows(tmp)
    for k in range(WCHUNKS):
        pltpu.sync_copy(tmp, acc.at[pl.ds(s * ROWS_PER_TILE + k * EB, EB)])
    plsc.subcore_barrier()

    def blk(j, carry):
        pltpu.async_copy(hs_hbm.at[src2d.at[j]], rows, sem).wait()
        pltpu.sync_copy(rows, acc.at[dst2d.at[j]], add=True)
        return carry

    lax.fori_loop(0, ETB, blk, 0)
    plsc.subcore_barrier()
    _writeout(acc, tmp, out_hbm, c, s)


_edge_kernel = pl.kernel(
    _edge_body,
    out_type=jax.ShapeDtypeStruct((NC * NP, H), jnp.float32),
    mesh=_mesh,
    scratch_types=[
        pltpu.VMEM((ETB, EB), jnp.int32),
        pltpu.VMEM((ETB, EB), jnp.int32),
        pltpu.VMEM((EB, H), jnp.float32),
        pltpu.VMEM((EB, H), jnp.float32),
        pltpu.VMEM_SHARED((NP, H), jnp.float32),
        pltpu.SemaphoreType.DMA,
    ],
)


def _dinv_from(degp_ref):
    deg = degp_ref[:NP] + degp_ref[NP:] + 1.0
    return lax.rsqrt(deg)


def _tc1_body(xp_ref, w1_ref, degp_ref, hs1_ref):
    dinv = _dinv_from(degp_ref)
    h1 = jnp.dot(xp_ref[...], w1_ref[...], preferred_element_type=jnp.float32)
    hs1_ref[...] = h1 * dinv


def _tc2_body(p1_ref, hs1_ref, degp_ref, b1_ref, w2_ref, hs2_ref):
    dinv = _dinv_from(degp_ref)
    hs1 = hs1_ref[...]
    y1 = (p1_ref[:NP] + p1_ref[NP:] + hs1) * dinv + b1_ref[...]
    y1 = jnp.maximum(y1, 0.0)
    h2 = jnp.dot(y1, w2_ref[...], preferred_element_type=jnp.float32)
    hs2_ref[...] = h2 * dinv


def _tc3_body(p2_ref, hs2_ref, degp_ref, b2_ref, out_ref):
    dinv = _dinv_from(degp_ref)
    z = (p2_ref[:NP] + p2_ref[NP:] + hs2_ref[...]) * dinv + b2_ref[...]
    zmax = jnp.max(z, axis=1, keepdims=True)
    ez = jnp.exp(z - zmax)
    lse = jnp.log(jnp.sum(ez, axis=1, keepdims=True))
    out_ref[...] = z - zmax - lse


_tc1 = pl.pallas_call(
    _tc1_body, out_shape=jax.ShapeDtypeStruct((NP, H), jnp.float32))
_tc2 = pl.pallas_call(
    _tc2_body, out_shape=jax.ShapeDtypeStruct((NP, H), jnp.float32))
_tc3 = pl.pallas_call(
    _tc3_body, out_shape=jax.ShapeDtypeStruct((NP, H), jnp.float32))


@jax.jit
def kernel(x, edge_index, W1, b1, W2, b2):
    src = edge_index[0].astype(jnp.int32)
    dst = edge_index[1].astype(jnp.int32)
    padidx = (jnp.arange(EPAD - E, dtype=jnp.int32) % (NP - N)) + N
    srcb = jnp.concatenate([src, padidx]).reshape(NBLK, EB)
    dstb = jnp.concatenate([dst, padidx]).reshape(NBLK, EB)
    xp = jnp.pad(x, ((0, NP - N), (0, 0)))

    degp = _deg_kernel(dstb)
    hs1 = _tc1(xp, W1, degp)
    p1 = _edge_kernel(hs1, srcb, dstb)
    hs2 = _tc2(p1, hs1, degp, b1.reshape(1, H), W2)
    p2 = _edge_kernel(hs2, srcb, dstb)
    out = _tc3(p2, hs2, degp, b2.reshape(1, H))
    return out[:N]


# speedup vs baseline: 34.8035x; 34.8035x over previous
import functools

import jax
import jax.numpy as jnp
from jax import lax
from jax.experimental import pallas as pl
from jax.experimental.pallas import tpu as pltpu
from jax.experimental.pallas import tpu_sc as plsc

N = 10000
NP = 10240
E = 320000
D = 128
H = 16

NC = 2
NS = 16
NW = NC * NS

EB = 128
ETB = 80
ET = EB * ETB
EPAD = ET * NW
NBLK = EPAD // EB

ROWS_PER_TILE = NP // NS
WCHUNKS = ROWS_PER_TILE // EB

_mesh = plsc.VectorSubcoreMesh(core_axis_name="c", subcore_axis_name="s")
_sc_params = pltpu.CompilerParams(use_tc_tiling_on_sc=False)


def _zero_rows(buf):
    z = jnp.zeros((H,), jnp.float32)
    for i in range(EB):
        buf[i] = z


def _fill_ones(buf):
    o = jnp.ones((H,), jnp.float32)
    for i in range(EB):
        buf[i] = o


def _writeout(acc, tmp, out_hbm, c, s):
    for k in range(WCHUNKS):
        off = s * ROWS_PER_TILE + k * EB
        pltpu.sync_copy(acc.at[pl.ds(off, EB)], tmp)
        pltpu.sync_copy(tmp, out_hbm.at[pl.ds(c * NP + off, EB)])


def _deg_body(dstb_hbm, out_hbm, dst2d, ones2d, tmp, acc):
    c = lax.axis_index("c")
    s = lax.axis_index("s")
    w = c * NS + s
    pltpu.sync_copy(dstb_hbm.at[pl.ds(w * ETB, ETB)], dst2d)
    _fill_ones(ones2d)
    _zero_rows(tmp)
    for k in range(WCHUNKS):
        pltpu.sync_copy(tmp, acc.at[pl.ds(s * ROWS_PER_TILE + k * EB, EB)])
    plsc.subcore_barrier()

    def blk(j, carry):
        pltpu.sync_copy(ones2d, acc.at[dst2d.at[j]], add=True)
        return carry

    lax.fori_loop(0, ETB, blk, 0)
    plsc.subcore_barrier()
    _writeout(acc, tmp, out_hbm, c, s)


_deg_kernel = pl.kernel(
    _deg_body,
    out_type=jax.ShapeDtypeStruct((NC * NP, H), jnp.float32),
    mesh=_mesh,
    compiler_params=_sc_params,
    scratch_types=[
        pltpu.VMEM((ETB, EB), jnp.int32),
        pltpu.VMEM((EB, H), jnp.float32),
        pltpu.VMEM((EB, H), jnp.float32),
        pltpu.VMEM_SHARED((NP, H), jnp.float32),
    ],
)


def _edge_body(hs_hbm, srcb_hbm, dstb_hbm, out_hbm, src2d, dst2d, rows, tmp,
               acc, sem):
    c = lax.axis_index("c")
    s = lax.axis_index("s")
    w = c * NS + s
    pltpu.sync_copy(srcb_hbm.at[pl.ds(w * ETB, ETB)], src2d)
    pltpu.sync_copy(dstb_hbm.at[pl.ds(w * ETB, ETB)], dst2d)
    _zero_rows(tmp)
    for k in range(WCHUNKS):
        pltpu.sync_copy(tmp, acc.at[pl.ds(s * ROWS_PER_TILE + k * EB, EB)])
    plsc.subcore_barrier()

    def blk(j, carry):
        pltpu.async_copy(hs_hbm.at[src2d.at[j]], rows, sem).wait()
        pltpu.sync_copy(rows, acc.at[dst2d.at[j]], add=True)
        return carry

    lax.fori_loop(0, ETB, blk, 0)
    plsc.subcore_barrier()
    _writeout(acc, tmp, out_hbm, c, s)


_edge_kernel = pl.kernel(
    _edge_body,
    out_type=jax.ShapeDtypeStruct((NC * NP, H), jnp.float32),
    mesh=_mesh,
    compiler_params=_sc_params,
    scratch_types=[
        pltpu.VMEM((ETB, EB), jnp.int32),
        pltpu.VMEM((ETB, EB), jnp.int32),
        pltpu.VMEM((EB, H), jnp.float32),
        pltpu.VMEM((EB, H), jnp.float32),
        pltpu.VMEM_SHARED((NP, H), jnp.float32),
        pltpu.SemaphoreType.DMA,
    ],
)


def _dinv_from(degp_ref):
    deg = degp_ref[:NP] + degp_ref[NP:] + 1.0
    return lax.rsqrt(deg)


def _tc1_body(xp_ref, w1_ref, degp_ref, hs1_ref):
    dinv = _dinv_from(degp_ref)
    h1 = jnp.dot(xp_ref[...], w1_ref[...], preferred_element_type=jnp.float32)
    hs1_ref[...] = h1 * dinv


def _tc2_body(p1_ref, hs1_ref, degp_ref, b1_ref, w2_ref, hs2_ref):
    dinv = _dinv_from(degp_ref)
    hs1 = hs1_ref[...]
    y1 = (p1_ref[:NP] + p1_ref[NP:] + hs1) * dinv + b1_ref[...]
    y1 = jnp.maximum(y1, 0.0)
    h2 = jnp.dot(y1, w2_ref[...], preferred_element_type=jnp.float32)
    hs2_ref[...] = h2 * dinv


def _tc3_body(p2_ref, hs2_ref, degp_ref, b2_ref, out_ref):
    dinv = _dinv_from(degp_ref)
    z = (p2_ref[:NP] + p2_ref[NP:] + hs2_ref[...]) * dinv + b2_ref[...]
    zmax = jnp.max(z, axis=1, keepdims=True)
    ez = jnp.exp(z - zmax)
    lse = jnp.log(jnp.sum(ez, axis=1, keepdims=True))
    out_ref[...] = z - zmax - lse


_tc1 = pl.pallas_call(
    _tc1_body, out_shape=jax.ShapeDtypeStruct((NP, H), jnp.float32))
_tc2 = pl.pallas_call(
    _tc2_body, out_shape=jax.ShapeDtypeStruct((NP, H), jnp.float32))
_tc3 = pl.pallas_call(
    _tc3_body, out_shape=jax.ShapeDtypeStruct((NP, H), jnp.float32))


@jax.jit
def kernel(x, edge_index, W1, b1, W2, b2):
    src = edge_index[0].astype(jnp.int32)
    dst = edge_index[1].astype(jnp.int32)
    padidx = (jnp.arange(EPAD - E, dtype=jnp.int32) % (NP - N)) + N
    srcb = jnp.concatenate([src, padidx]).reshape(NBLK, EB)
    dstb = jnp.concatenate([dst, padidx]).reshape(NBLK, EB)
    xp = jnp.pad(x, ((0, NP - N), (0, 0)))

    degp = _deg_kernel(dstb)
    hs1 = _tc1(xp, W1, degp)
    p1 = _edge_kernel(hs1, srcb, dstb)
    hs2 = _tc2(p1, hs1, degp, b1.reshape(1, H), W2)
    p2 = _edge_kernel(hs2, srcb, dstb)
    out = _tc3(p2, hs2, degp, b2.reshape(1, H))
    return out[:N]


# speedup vs baseline: 56.0625x; 1.6108x over previous
import functools

import jax
import jax.numpy as jnp
from jax import lax
from jax.experimental import pallas as pl
from jax.experimental.pallas import tpu as pltpu
from jax.experimental.pallas import tpu_sc as plsc

N = 10000
NP = 10240
E = 320000
D = 128
H = 16

NC = 2
NS = 16
NW = NC * NS

EB = 128
ETB = 80
ET = EB * ETB
EPAD = ET * NW
NBLK = EPAD // EB

ROWS_PER_TILE = NP // NS
WCHUNKS = ROWS_PER_TILE // EB

_mesh = plsc.VectorSubcoreMesh(core_axis_name="c", subcore_axis_name="s")
_sc_params = pltpu.CompilerParams(use_tc_tiling_on_sc=False)


def _zero_rows(buf):
    z = jnp.zeros((H,), jnp.float32)
    for i in range(EB):
        buf[i] = z


def _fill_ones(buf):
    o = jnp.ones((H,), jnp.float32)
    for i in range(EB):
        buf[i] = o


def _writeout(acc, tmp, out_hbm, c, s):
    for k in range(WCHUNKS):
        off = s * ROWS_PER_TILE + k * EB
        pltpu.sync_copy(acc.at[pl.ds(off, EB)], tmp)
        pltpu.sync_copy(tmp, out_hbm.at[pl.ds(c * NP + off, EB)])


def _deg_body(dstb_hbm, out_hbm, dst2d, ones2d, tmp, acc, sem):
    c = lax.axis_index("c")
    s = lax.axis_index("s")
    w = c * NS + s
    pltpu.sync_copy(dstb_hbm.at[pl.ds(w * ETB, ETB)], dst2d)
    _fill_ones(ones2d)
    _zero_rows(tmp)
    for k in range(WCHUNKS):
        pltpu.sync_copy(tmp, acc.at[pl.ds(s * ROWS_PER_TILE + k * EB, EB)])
    plsc.subcore_barrier()

    def blk(j, carry):
        pltpu.async_copy(ones2d, acc.at[dst2d.at[j]], sem, add=True)
        return carry

    lax.fori_loop(0, ETB, blk, 0)

    def drain(j, carry):
        pltpu.make_async_copy(ones2d, acc.at[dst2d.at[j]], sem).wait()
        return carry

    lax.fori_loop(0, ETB, drain, 0)
    plsc.subcore_barrier()
    _writeout(acc, tmp, out_hbm, c, s)


_deg_kernel = pl.kernel(
    _deg_body,
    out_type=jax.ShapeDtypeStruct((NC * NP, H), jnp.float32),
    mesh=_mesh,
    compiler_params=_sc_params,
    scratch_types=[
        pltpu.VMEM((ETB, EB), jnp.int32),
        pltpu.VMEM((EB, H), jnp.float32),
        pltpu.VMEM((EB, H), jnp.float32),
        pltpu.VMEM_SHARED((NP, H), jnp.float32),
        pltpu.SemaphoreType.DMA,
    ],
)


NBUF = 4
NGRP = ETB // NBUF


def _edge_body(hs_hbm, srcb_hbm, dstb_hbm, out_hbm, src2d, dst2d, rows, tmp,
               acc, gsem, ssem):
    c = lax.axis_index("c")
    s = lax.axis_index("s")
    w = c * NS + s
    pltpu.sync_copy(srcb_hbm.at[pl.ds(w * ETB, ETB)], src2d)
    pltpu.sync_copy(dstb_hbm.at[pl.ds(w * ETB, ETB)], dst2d)
    _zero_rows(tmp)
    for k in range(WCHUNKS):
        pltpu.sync_copy(tmp, acc.at[pl.ds(s * ROWS_PER_TILE + k * EB, EB)])
    plsc.subcore_barrier()

    for b in range(NBUF):
        pltpu.async_copy(hs_hbm.at[src2d.at[b]], rows.at[b], gsem.at[b])

    def grp(g, carry):
        for b in range(NBUF):
            j = g * NBUF + b
            pltpu.make_async_copy(
                hs_hbm.at[src2d.at[j]], rows.at[b], gsem.at[b]).wait()
            pltpu.async_copy(rows.at[b], acc.at[dst2d.at[j]], ssem.at[b],
                             add=True)
            pltpu.make_async_copy(
                rows.at[b], acc.at[dst2d.at[j]], ssem.at[b]).wait()
            nj = j + NBUF

            @pl.when(nj < ETB)
            def _():
                pltpu.async_copy(hs_hbm.at[src2d.at[nj]], rows.at[b],
                                 gsem.at[b])
        return carry

    lax.fori_loop(0, NGRP, grp, 0)
    plsc.subcore_barrier()
    _writeout(acc, tmp, out_hbm, c, s)


_edge_kernel = pl.kernel(
    _edge_body,
    out_type=jax.ShapeDtypeStruct((NC * NP, H), jnp.float32),
    mesh=_mesh,
    compiler_params=_sc_params,
    scratch_types=[
        pltpu.VMEM((ETB, EB), jnp.int32),
        pltpu.VMEM((ETB, EB), jnp.int32),
        pltpu.VMEM((NBUF, EB, H), jnp.float32),
        pltpu.VMEM((EB, H), jnp.float32),
        pltpu.VMEM_SHARED((NP, H), jnp.float32),
        pltpu.SemaphoreType.DMA((NBUF,)),
        pltpu.SemaphoreType.DMA((NBUF,)),
    ],
)


def _dinv_from(degp_ref):
    deg = degp_ref[:NP] + degp_ref[NP:] + 1.0
    return lax.rsqrt(deg)


def _tc1_body(xp_ref, w1_ref, degp_ref, hs1_ref):
    dinv = _dinv_from(degp_ref)
    h1 = jnp.dot(xp_ref[...], w1_ref[...], preferred_element_type=jnp.float32)
    hs1_ref[...] = h1 * dinv


def _tc2_body(p1_ref, hs1_ref, degp_ref, b1_ref, w2_ref, hs2_ref):
    dinv = _dinv_from(degp_ref)
    hs1 = hs1_ref[...]
    y1 = (p1_ref[:NP] + p1_ref[NP:] + hs1) * dinv + b1_ref[...]
    y1 = jnp.maximum(y1, 0.0)
    h2 = jnp.dot(y1, w2_ref[...], preferred_element_type=jnp.float32)
    hs2_ref[...] = h2 * dinv


def _tc3_body(p2_ref, hs2_ref, degp_ref, b2_ref, out_ref):
    dinv = _dinv_from(degp_ref)
    z = (p2_ref[:NP] + p2_ref[NP:] + hs2_ref[...]) * dinv + b2_ref[...]
    zmax = jnp.max(z, axis=1, keepdims=True)
    ez = jnp.exp(z - zmax)
    lse = jnp.log(jnp.sum(ez, axis=1, keepdims=True))
    out_ref[...] = z - zmax - lse


_tc1 = pl.pallas_call(
    _tc1_body, out_shape=jax.ShapeDtypeStruct((NP, H), jnp.float32))
_tc2 = pl.pallas_call(
    _tc2_body, out_shape=jax.ShapeDtypeStruct((NP, H), jnp.float32))
_tc3 = pl.pallas_call(
    _tc3_body, out_shape=jax.ShapeDtypeStruct((NP, H), jnp.float32))


@jax.jit
def kernel(x, edge_index, W1, b1, W2, b2):
    src = edge_index[0].astype(jnp.int32)
    dst = edge_index[1].astype(jnp.int32)
    padidx = (jnp.arange(EPAD - E, dtype=jnp.int32) % (NP - N)) + N
    srcb = jnp.concatenate([src, padidx]).reshape(NBLK, EB)
    dstb = jnp.concatenate([dst, padidx]).reshape(NBLK, EB)
    xp = jnp.pad(x, ((0, NP - N), (0, 0)))

    degp = _deg_kernel(dstb)
    hs1 = _tc1(xp, W1, degp)
    p1 = _edge_kernel(hs1, srcb, dstb)
    hs2 = _tc2(p1, hs1, degp, b1.reshape(1, H), W2)
    p2 = _edge_kernel(hs2, srcb, dstb)
    out = _tc3(p2, hs2, degp, b2.reshape(1, H))
    return out[:N]


# speedup vs baseline: 81.0735x; 1.4461x over previous
import functools

import jax
import jax.numpy as jnp
from jax import lax
from jax.experimental import pallas as pl
from jax.experimental.pallas import tpu as pltpu
from jax.experimental.pallas import tpu_sc as plsc

N = 10000
NP = 10240
E = 320000
D = 128
H = 16

NC = 2
NS = 16
NW = NC * NS

EB = 128
ETB = 80
ET = EB * ETB
EPAD = ET * NW
NBLK = EPAD // EB

ROWS_PER_TILE = NP // NS
WCHUNKS = ROWS_PER_TILE // EB

_mesh = plsc.VectorSubcoreMesh(core_axis_name="c", subcore_axis_name="s")
_sc_params = pltpu.CompilerParams(use_tc_tiling_on_sc=False)


def _zero_rows(buf):
    z = jnp.zeros((H,), jnp.float32)
    for i in range(EB):
        buf[i] = z


def _fill_ones(buf):
    o = jnp.ones((H,), jnp.float32)
    for i in range(EB):
        buf[i] = o


def _writeout(acc, tmp, out_hbm, c, s):
    for k in range(WCHUNKS):
        off = s * ROWS_PER_TILE + k * EB
        pltpu.sync_copy(acc.at[pl.ds(off, EB)], tmp)
        pltpu.sync_copy(tmp, out_hbm.at[pl.ds(c * NP + off, EB)])


def _deg_body(dstb_hbm, out_hbm, dst2d, ones2d, tmp, acc, sem):
    c = lax.axis_index("c")
    s = lax.axis_index("s")
    w = c * NS + s
    pltpu.sync_copy(dstb_hbm.at[pl.ds(w * ETB, ETB)], dst2d)
    _fill_ones(ones2d)
    _zero_rows(tmp)
    for k in range(WCHUNKS):
        pltpu.sync_copy(tmp, acc.at[pl.ds(s * ROWS_PER_TILE + k * EB, EB)])
    plsc.subcore_barrier()

    def blk(j, carry):
        pltpu.async_copy(ones2d, acc.at[dst2d.at[j]], sem, add=True)
        return carry

    lax.fori_loop(0, ETB, blk, 0)

    def drain(j, carry):
        pltpu.make_async_copy(ones2d, acc.at[dst2d.at[j]], sem).wait()
        return carry

    lax.fori_loop(0, ETB, drain, 0)
    plsc.subcore_barrier()
    _writeout(acc, tmp, out_hbm, c, s)


_deg_kernel = pl.kernel(
    _deg_body,
    out_type=jax.ShapeDtypeStruct((NC * NP, H), jnp.float32),
    mesh=_mesh,
    compiler_params=_sc_params,
    scratch_types=[
        pltpu.VMEM((ETB, EB), jnp.int32),
        pltpu.VMEM((EB, H), jnp.float32),
        pltpu.VMEM((EB, H), jnp.float32),
        pltpu.VMEM_SHARED((NP, H), jnp.float32),
        pltpu.SemaphoreType.DMA,
    ],
)


NBUF = 4
NGRP = ETB // NBUF


def _edge_body(hs_hbm, srcb_hbm, dstb_hbm, out_hbm, src2d, dst2d, rows, tmp,
               acc, gsem, ssem):
    c = lax.axis_index("c")
    s = lax.axis_index("s")
    w = c * NS + s
    pltpu.sync_copy(srcb_hbm.at[pl.ds(w * ETB, ETB)], src2d)
    pltpu.sync_copy(dstb_hbm.at[pl.ds(w * ETB, ETB)], dst2d)
    _zero_rows(tmp)
    for k in range(WCHUNKS):
        pltpu.sync_copy(tmp, acc.at[pl.ds(s * ROWS_PER_TILE + k * EB, EB)])
    plsc.subcore_barrier()

    for b in range(NBUF):
        pltpu.async_copy(hs_hbm.at[src2d.at[b]], rows.at[b], gsem.at[b])

    def grp(g, carry):
        for b in range(NBUF):
            j = g * NBUF + b
            pltpu.make_async_copy(
                hs_hbm.at[src2d.at[j]], rows.at[b], gsem.at[b]).wait()
            pltpu.async_copy(rows.at[b], acc.at[dst2d.at[j]], ssem.at[b],
                             add=True)
            pltpu.make_async_copy(
                rows.at[b], acc.at[dst2d.at[j]], ssem.at[b]).wait()
            nj = j + NBUF

            @pl.when(nj < ETB)
            def _():
                pltpu.async_copy(hs_hbm.at[src2d.at[nj]], rows.at[b],
                                 gsem.at[b])
        return carry

    lax.fori_loop(0, NGRP, grp, 0)
    plsc.subcore_barrier()
    _writeout(acc, tmp, out_hbm, c, s)


_edge_kernel = pl.kernel(
    _edge_body,
    out_type=jax.ShapeDtypeStruct((NC * NP, H), jnp.float32),
    mesh=_mesh,
    compiler_params=_sc_params,
    scratch_types=[
        pltpu.VMEM((ETB, EB), jnp.int32),
        pltpu.VMEM((ETB, EB), jnp.int32),
        pltpu.VMEM((NBUF, EB, H), jnp.float32),
        pltpu.VMEM((EB, H), jnp.float32),
        pltpu.VMEM_SHARED((NP, H), jnp.float32),
        pltpu.SemaphoreType.DMA((NBUF,)),
        pltpu.SemaphoreType.DMA((NBUF,)),
    ],
)


NPP = NP // 8


def _dinv_packed(degp_ref):
    deg = degp_ref[:NPP] + degp_ref[NPP:] + 1.0
    return lax.rsqrt(deg)


def _tc1_body(xp8_ref, w1b_ref, degp_ref, hs1_ref):
    dinv = _dinv_packed(degp_ref)
    h1p = jnp.dot(xp8_ref[...], w1b_ref[...],
                  preferred_element_type=jnp.float32)
    hs1_ref[...] = h1p * dinv


def _tc2_body(p1_ref, hs1_ref, degp_ref, b1_ref, w2b_ref, hs2_ref):
    dinv = _dinv_packed(degp_ref)
    y1 = (p1_ref[:NPP] + p1_ref[NPP:] + hs1_ref[...]) * dinv + b1_ref[...]
    y1 = jnp.maximum(y1, 0.0)
    h2p = jnp.dot(y1, w2b_ref[...], preferred_element_type=jnp.float32)
    hs2_ref[...] = h2p * dinv


def _tc3_body(p2_ref, hs2_ref, degp_ref, b2_ref, bsum_ref, out_ref):
    dinv = _dinv_packed(degp_ref)
    zp = (p2_ref[:NPP] + p2_ref[NPP:] + hs2_ref[...]) * dinv + b2_ref[...]
    m = jnp.max(zp, axis=1, keepdims=True)
    ez = jnp.exp(zp - m)
    s = jnp.dot(ez, bsum_ref[...], preferred_element_type=jnp.float32)
    out_ref[...] = zp - m - jnp.log(s)


_tc1 = pl.pallas_call(
    _tc1_body, out_shape=jax.ShapeDtypeStruct((NPP, 128), jnp.float32))
_tc2 = pl.pallas_call(
    _tc2_body, out_shape=jax.ShapeDtypeStruct((NPP, 128), jnp.float32))
_tc3 = pl.pallas_call(
    _tc3_body, out_shape=jax.ShapeDtypeStruct((NPP, 128), jnp.float32))


@jax.jit
def kernel(x, edge_index, W1, b1, W2, b2):
    src = edge_index[0].astype(jnp.int32)
    dst = edge_index[1].astype(jnp.int32)
    padidx = (jnp.arange(EPAD - E, dtype=jnp.int32) % (NP - N)) + N
    srcb = jnp.concatenate([src, padidx]).reshape(NBLK, EB)
    dstb = jnp.concatenate([dst, padidx]).reshape(NBLK, EB)
    xp8 = jnp.pad(x, ((0, NP - N), (0, 0))).reshape(NPP, 8 * D)
    eye8 = jnp.eye(8, dtype=jnp.float32)
    w1b = jnp.kron(eye8, W1)
    w2b = jnp.kron(eye8, W2)
    bsum = jnp.kron(eye8, jnp.ones((H, H), jnp.float32))
    b1p = jnp.tile(b1, 8).reshape(1, 128)
    b2p = jnp.tile(b2, 8).reshape(1, 128)

    degp = _deg_kernel(dstb).reshape(2 * NPP, 128)
    hs1p = _tc1(xp8, w1b, degp)
    p1 = _edge_kernel(hs1p.reshape(NP, H), srcb, dstb).reshape(2 * NPP, 128)
    hs2p = _tc2(p1, hs1p, degp, b1p, w2b)
    p2 = _edge_kernel(hs2p.reshape(NP, H), srcb, dstb).reshape(2 * NPP, 128)
    outp = _tc3(p2, hs2p, degp, b2p, bsum)
    return outp.reshape(NP, H)[:N]


# speedup vs baseline: 83.6161x; 1.0314x over previous
import functools

import jax
import jax.numpy as jnp
from jax import lax
from jax.experimental import pallas as pl
from jax.experimental.pallas import tpu as pltpu
from jax.experimental.pallas import tpu_sc as plsc

N = 10000
NP = 10240
E = 320000
D = 128
H = 16

NC = 2
NS = 16
NW = NC * NS

EB = 128
ETB = 80
ET = EB * ETB
EPAD = ET * NW
NBLK = EPAD // EB

ROWS_PER_TILE = NP // NS
WCHUNKS = ROWS_PER_TILE // EB

_mesh = plsc.VectorSubcoreMesh(core_axis_name="c", subcore_axis_name="s")
_sc_params = pltpu.CompilerParams(use_tc_tiling_on_sc=False)


def _zero_rows(buf):
    z = jnp.zeros((H,), jnp.float32)
    for i in range(EB):
        buf[i] = z


def _fill_ones(buf):
    o = jnp.ones((H,), jnp.float32)
    for i in range(EB):
        buf[i] = o


def _writeout(acc, tmp, out_hbm, c, s):
    for k in range(WCHUNKS):
        off = s * ROWS_PER_TILE + k * EB
        pltpu.sync_copy(acc.at[pl.ds(off, EB)], tmp)
        pltpu.sync_copy(tmp, out_hbm.at[pl.ds(c * NP + off, EB)])


def _deg_body(dstb_hbm, out_hbm, dst2d, ones2d, tmp, acc, sem):
    c = lax.axis_index("c")
    s = lax.axis_index("s")
    w = c * NS + s
    pltpu.sync_copy(dstb_hbm.at[pl.ds(w * ETB, ETB)], dst2d)
    _fill_ones(ones2d)
    _zero_rows(tmp)
    for k in range(WCHUNKS):
        pltpu.sync_copy(tmp, acc.at[pl.ds(s * ROWS_PER_TILE + k * EB, EB)])
    plsc.subcore_barrier()

    def blk(j, carry):
        pltpu.async_copy(ones2d, acc.at[dst2d.at[j]], sem, add=True)
        return carry

    lax.fori_loop(0, ETB, blk, 0)

    def drain(j, carry):
        pltpu.make_async_copy(ones2d, acc.at[dst2d.at[j]], sem).wait()
        return carry

    lax.fori_loop(0, ETB, drain, 0)
    plsc.subcore_barrier()
    _writeout(acc, tmp, out_hbm, c, s)


_deg_kernel = pl.kernel(
    _deg_body,
    out_type=jax.ShapeDtypeStruct((NC * NP, H), jnp.float32),
    mesh=_mesh,
    compiler_params=_sc_params,
    scratch_types=[
        pltpu.VMEM((ETB, EB), jnp.int32),
        pltpu.VMEM((EB, H), jnp.float32),
        pltpu.VMEM((EB, H), jnp.float32),
        pltpu.VMEM_SHARED((NP, H), jnp.float32),
        pltpu.SemaphoreType.DMA,
    ],
)


LOOK = 4
SLOTS = 8
NGRP = ETB // SLOTS


def _edge_body(hs_hbm, srcb_hbm, dstb_hbm, out_hbm, src2d, dst2d, rows, tmp,
               acc, gsem, ssem):
    c = lax.axis_index("c")
    s = lax.axis_index("s")
    w = c * NS + s
    pltpu.sync_copy(srcb_hbm.at[pl.ds(w * ETB, ETB)], src2d)
    pltpu.sync_copy(dstb_hbm.at[pl.ds(w * ETB, ETB)], dst2d)
    _zero_rows(tmp)
    for k in range(WCHUNKS):
        pltpu.sync_copy(tmp, acc.at[pl.ds(s * ROWS_PER_TILE + k * EB, EB)])
    plsc.subcore_barrier()

    def block(j, b, wait_prev, fire):
        pltpu.make_async_copy(
            hs_hbm.at[src2d.at[j]], rows.at[b], gsem.at[b]).wait()
        pltpu.async_copy(rows.at[b], acc.at[dst2d.at[j]], ssem.at[b],
                         add=True)
        if fire:
            nb = (b + LOOK) % SLOTS
            if wait_prev:
                pltpu.make_async_copy(
                    rows.at[nb], acc.at[dst2d.at[j]], ssem.at[nb]).wait()
            pltpu.async_copy(hs_hbm.at[src2d.at[j + LOOK]], rows.at[nb],
                             gsem.at[nb])

    for b in range(LOOK):
        pltpu.async_copy(hs_hbm.at[src2d.at[b]], rows.at[b], gsem.at[b])
    for b in range(SLOTS):
        block(b, b, wait_prev=(b >= LOOK), fire=True)

    def grp(g, carry):
        for b in range(SLOTS):
            block(g * SLOTS + b, b, wait_prev=True, fire=True)
        return carry

    lax.fori_loop(1, NGRP - 1, grp, 0)

    for b in range(SLOTS):
        j = (NGRP - 1) * SLOTS + b
        block(j, b, wait_prev=True, fire=(b < LOOK))
    for b in range(SLOTS):
        pltpu.make_async_copy(
            rows.at[b], acc.at[dst2d.at[0]], ssem.at[b]).wait()

    plsc.subcore_barrier()
    _writeout(acc, tmp, out_hbm, c, s)


_edge_kernel = pl.kernel(
    _edge_body,
    out_type=jax.ShapeDtypeStruct((NC * NP, H), jnp.float32),
    mesh=_mesh,
    compiler_params=_sc_params,
    scratch_types=[
        pltpu.VMEM((ETB, EB), jnp.int32),
        pltpu.VMEM((ETB, EB), jnp.int32),
        pltpu.VMEM((SLOTS, EB, H), jnp.float32),
        pltpu.VMEM((EB, H), jnp.float32),
        pltpu.VMEM_SHARED((NP, H), jnp.float32),
        pltpu.SemaphoreType.DMA((SLOTS,)),
        pltpu.SemaphoreType.DMA((SLOTS,)),
    ],
)


NPP = NP // 8


def _dinv_packed(degp_ref):
    deg = degp_ref[:NPP] + degp_ref[NPP:] + 1.0
    return lax.rsqrt(deg)


def _tc1_body(xp8_ref, w1b_ref, degp_ref, hs1_ref):
    dinv = _dinv_packed(degp_ref)
    h1p = jnp.dot(xp8_ref[...], w1b_ref[...],
                  preferred_element_type=jnp.float32)
    hs1_ref[...] = h1p * dinv


def _tc2_body(p1_ref, hs1_ref, degp_ref, b1_ref, w2b_ref, hs2_ref):
    dinv = _dinv_packed(degp_ref)
    y1 = (p1_ref[:NPP] + p1_ref[NPP:] + hs1_ref[...]) * dinv + b1_ref[...]
    y1 = jnp.maximum(y1, 0.0)
    h2p = jnp.dot(y1, w2b_ref[...], preferred_element_type=jnp.float32)
    hs2_ref[...] = h2p * dinv


def _tc3_body(p2_ref, hs2_ref, degp_ref, b2_ref, bsum_ref, out_ref):
    dinv = _dinv_packed(degp_ref)
    zp = (p2_ref[:NPP] + p2_ref[NPP:] + hs2_ref[...]) * dinv + b2_ref[...]
    m = jnp.max(zp, axis=1, keepdims=True)
    ez = jnp.exp(zp - m)
    s = jnp.dot(ez, bsum_ref[...], preferred_element_type=jnp.float32)
    out_ref[...] = zp - m - jnp.log(s)


_tc1 = pl.pallas_call(
    _tc1_body, out_shape=jax.ShapeDtypeStruct((NPP, 128), jnp.float32))
_tc2 = pl.pallas_call(
    _tc2_body, out_shape=jax.ShapeDtypeStruct((NPP, 128), jnp.float32))
_tc3 = pl.pallas_call(
    _tc3_body, out_shape=jax.ShapeDtypeStruct((NPP, 128), jnp.float32))


@jax.jit
def kernel(x, edge_index, W1, b1, W2, b2):
    src = edge_index[0].astype(jnp.int32)
    dst = edge_index[1].astype(jnp.int32)
    padidx = (jnp.arange(EPAD - E, dtype=jnp.int32) % (NP - N)) + N
    srcb = jnp.concatenate([src, padidx]).reshape(NBLK, EB)
    dstb = jnp.concatenate([dst, padidx]).reshape(NBLK, EB)
    xp8 = jnp.pad(x, ((0, NP - N), (0, 0))).reshape(NPP, 8 * D)
    eye8 = jnp.eye(8, dtype=jnp.float32)
    w1b = jnp.kron(eye8, W1)
    w2b = jnp.kron(eye8, W2)
    bsum = jnp.kron(eye8, jnp.ones((H, H), jnp.float32))
    b1p = jnp.tile(b1, 8).reshape(1, 128)
    b2p = jnp.tile(b2, 8).reshape(1, 128)

    degp = _deg_kernel(dstb).reshape(2 * NPP, 128)
    hs1p = _tc1(xp8, w1b, degp)
    p1 = _edge_kernel(hs1p.reshape(NP, H), srcb, dstb).reshape(2 * NPP, 128)
    hs2p = _tc2(p1, hs1p, degp, b1p, w2b)
    p2 = _edge_kernel(hs2p.reshape(NP, H), srcb, dstb).reshape(2 * NPP, 128)
    outp = _tc3(p2, hs2p, degp, b2p, bsum)
    return outp.reshape(NP, H)[:N]


# speedup vs baseline: 97.8741x; 1.1705x over previous
import functools

import jax
import jax.numpy as jnp
from jax import lax
from jax.experimental import pallas as pl
from jax.experimental.pallas import tpu as pltpu
from jax.experimental.pallas import tpu_sc as plsc

N = 10000
NP = 10240
NPP = NP // 8
E = 320000
D = 128
H = 16

NC = 2
NS = 16
NW = NC * NS

EB = 128
ETB = 80
NBLK = E // EB
PADB = NW * ETB - NBLK

NODES_PER_TILE = NP // NS
WCHUNKS = NODES_PER_TILE // EB

LOOK = 4
SLOTS = 8
NGRP = ETB // SLOTS

_mesh = plsc.VectorSubcoreMesh(core_axis_name="c", subcore_axis_name="s")
_sc_params = pltpu.CompilerParams(use_tc_tiling_on_sc=False)


def _zero_rows(buf, n):
    z = jnp.zeros((H,), jnp.float32)
    for i in range(n):
        buf[i] = z


def _load_indices(ei3_hbm, pad3_hbm, eiv, w):
    @pl.when(w < NW - 1)
    def _():
        pltpu.sync_copy(ei3_hbm.at[pl.ds(w * ETB, ETB)], eiv)

    @pl.when(w == NW - 1)
    def _():
        real = NBLK - (NW - 1) * ETB
        pltpu.sync_copy(ei3_hbm.at[pl.ds((NW - 1) * ETB, real)],
                        eiv.at[pl.ds(0, real)])
        pltpu.sync_copy(pad3_hbm, eiv.at[pl.ds(real, PADB)])


def _deg_body(ei3_hbm, pad3_hbm, out_hbm, eiv, ones2d, tmp, acc, sem):
    c = lax.axis_index("c")
    s = lax.axis_index("s")
    w = c * NS + s
    _load_indices(ei3_hbm, pad3_hbm, eiv, w)
    o16 = jnp.ones((H,), jnp.float32)
    for i in range(EB):
        ones2d[i] = o16
    _zero_rows(tmp, EB)
    for k in range(WCHUNKS):
        pltpu.sync_copy(tmp, acc.at[pl.ds(s * NODES_PER_TILE + k * EB, EB)])
    plsc.subcore_barrier()

    def blk(j, carry):
        pltpu.async_copy(ones2d, acc.at[eiv.at[j, 1]], sem, add=True)
        return carry

    lax.fori_loop(0, ETB, blk, 0)

    def drain(j, carry):
        pltpu.make_async_copy(ones2d, acc.at[eiv.at[j, 1]], sem).wait()
        return carry

    lax.fori_loop(0, ETB, drain, 0)
    plsc.subcore_barrier()
    for k in range(WCHUNKS):
        off = s * NODES_PER_TILE + k * EB
        pltpu.sync_copy(acc.at[pl.ds(off, EB)], tmp)
        pltpu.sync_copy(tmp, out_hbm.at[pl.ds(c * NP + off, EB)])


_deg_kernel = pl.kernel(
    _deg_body,
    out_type=jax.ShapeDtypeStruct((NC * NP, H), jnp.float32),
    mesh=_mesh,
    compiler_params=_sc_params,
    scratch_types=[
        pltpu.VMEM((ETB, 2, EB), jnp.int32),
        pltpu.VMEM((EB, H), jnp.float32),
        pltpu.VMEM((EB, H), jnp.float32),
        pltpu.VMEM_SHARED((NP, H), jnp.float32),
        pltpu.SemaphoreType.DMA,
    ],
)


def _edge_body(hs_hbm, ei3_hbm, pad3_hbm, out_hbm, eiv, rows, tmp, acc,
               gsem, ssem):
    c = lax.axis_index("c")
    s = lax.axis_index("s")
    w = c * NS + s
    _load_indices(ei3_hbm, pad3_hbm, eiv, w)
    _zero_rows(tmp, EB)
    for k in range(WCHUNKS):
        pltpu.sync_copy(tmp, acc.at[pl.ds(s * NODES_PER_TILE + k * EB, EB)])
    plsc.subcore_barrier()

    def block(j, b, wait_prev, fire):
        pltpu.make_async_copy(
            hs_hbm.at[eiv.at[j, 0]], rows.at[b], gsem.at[b]).wait()
        pltpu.async_copy(rows.at[b], acc.at[eiv.at[j, 1]], ssem.at[b],
                         add=True)
        if fire:
            nb = (b + LOOK) % SLOTS
            if wait_prev:
                pltpu.make_async_copy(
                    rows.at[nb], acc.at[eiv.at[j, 1]], ssem.at[nb]).wait()
            pltpu.async_copy(hs_hbm.at[eiv.at[j + LOOK, 0]], rows.at[nb],
                             gsem.at[nb])

    for b in range(LOOK):
        pltpu.async_copy(hs_hbm.at[eiv.at[b, 0]], rows.at[b], gsem.at[b])
    for b in range(SLOTS):
        block(b, b, wait_prev=(b >= LOOK), fire=True)

    def grp(g, carry):
        for b in range(SLOTS):
            block(g * SLOTS + b, b, wait_prev=True, fire=True)
        return carry

    lax.fori_loop(1, NGRP - 1, grp, 0)

    for b in range(SLOTS):
        j = (NGRP - 1) * SLOTS + b
        block(j, b, wait_prev=True, fire=(b < LOOK))
    for b in range(SLOTS):
        pltpu.make_async_copy(
            rows.at[b], acc.at[eiv.at[0, 1]], ssem.at[b]).wait()

    plsc.subcore_barrier()
    for k in range(WCHUNKS):
        off = s * NODES_PER_TILE + k * EB
        pltpu.sync_copy(acc.at[pl.ds(off, EB)], tmp)
        pltpu.sync_copy(tmp, out_hbm.at[pl.ds(c * NP + off, EB)])


_edge_kernel = pl.kernel(
    _edge_body,
    out_type=jax.ShapeDtypeStruct((NC * NP, H), jnp.float32),
    mesh=_mesh,
    compiler_params=_sc_params,
    scratch_types=[
        pltpu.VMEM((ETB, 2, EB), jnp.int32),
        pltpu.VMEM((SLOTS, EB, H), jnp.float32),
        pltpu.VMEM((EB, H), jnp.float32),
        pltpu.VMEM_SHARED((NP, H), jnp.float32),
        pltpu.SemaphoreType.DMA((SLOTS,)),
        pltpu.SemaphoreType.DMA((SLOTS,)),
    ],
)


def _dinv_packed(degp_ref):
    deg = degp_ref[:NPP] + degp_ref[NPP:] + 1.0
    return lax.rsqrt(deg)


def _tc1a_body(xp8_ref, w1b_ref, h1_ref):
    h1_ref[...] = jnp.dot(xp8_ref[...], w1b_ref[...],
                          preferred_element_type=jnp.float32)


def _tc1b_body(h1_ref, degp_ref, hs1_ref):
    hs1_ref[...] = h1_ref[...] * _dinv_packed(degp_ref)


def _tc2_body(p1_ref, hs1_ref, degp_ref, b1_ref, w2b_ref, hs2_ref):
    dinv = _dinv_packed(degp_ref)
    y1 = (p1_ref[:NPP] + p1_ref[NPP:] + hs1_ref[...]) * dinv + b1_ref[...]
    y1 = jnp.maximum(y1, 0.0)
    h2p = jnp.dot(y1, w2b_ref[...], preferred_element_type=jnp.float32)
    hs2_ref[...] = h2p * dinv


def _tc3_body(p2_ref, hs2_ref, degp_ref, b2_ref, bsum_ref, out_ref):
    dinv = _dinv_packed(degp_ref)
    zp = (p2_ref[:NPP] + p2_ref[NPP:] + hs2_ref[...]) * dinv + b2_ref[...]
    m = jnp.max(zp, axis=1, keepdims=True)
    ez = jnp.exp(zp - m)
    s = jnp.dot(ez, bsum_ref[...], preferred_element_type=jnp.float32)
    out_ref[...] = (zp - m - jnp.log(s))[:N // 8]


_tc1a = pl.pallas_call(
    _tc1a_body, out_shape=jax.ShapeDtypeStruct((NPP, 128), jnp.float32))
_tc1b = pl.pallas_call(
    _tc1b_body, out_shape=jax.ShapeDtypeStruct((NPP, 128), jnp.float32))
_tc2 = pl.pallas_call(
    _tc2_body, out_shape=jax.ShapeDtypeStruct((NPP, 128), jnp.float32))
_tc3 = pl.pallas_call(
    _tc3_body, out_shape=jax.ShapeDtypeStruct((N // 8, 128), jnp.float32))


@jax.jit
def kernel(x, edge_index, W1, b1, W2, b2):
    ei3 = edge_index.astype(jnp.int32).reshape(2, NBLK, EB).transpose(1, 0, 2)
    pad3 = (N + jnp.arange(PADB * 2 * EB, dtype=jnp.int32) % (NP - N)
            ).reshape(PADB, 2, EB)
    xp8 = jnp.pad(x, ((0, NP - N), (0, 0))).reshape(NPP, 8 * D)
    eye8 = jnp.eye(8, dtype=jnp.float32)
    w1b = jnp.kron(eye8, W1)
    w2b = jnp.kron(eye8, W2)
    bsum = jnp.kron(eye8, jnp.ones((H, H), jnp.float32))
    b1p = jnp.tile(b1, 8).reshape(1, 128)
    b2p = jnp.tile(b2, 8).reshape(1, 128)

    degp = _deg_kernel(ei3, pad3).reshape(2 * NPP, 128)
    h1p = _tc1a(xp8, w1b)
    hs1p = _tc1b(h1p, degp)
    p1 = _edge_kernel(hs1p.reshape(NP, H), ei3, pad3).reshape(2 * NPP, 128)
    hs2p = _tc2(p1, hs1p, degp, b1p, w2b)
    p2 = _edge_kernel(hs2p.reshape(NP, H), ei3, pad3).reshape(2 * NPP, 128)
    outp = _tc3(p2, hs2p, degp, b2p, bsum)
    return outp.reshape(N, H)


# speedup vs baseline: 102.8867x; 1.0512x over previous
import functools

import jax
import jax.numpy as jnp
from jax import lax
from jax.experimental import pallas as pl
from jax.experimental.pallas import tpu as pltpu
from jax.experimental.pallas import tpu_sc as plsc

N = 10000
NP = 10240
NPP = NP // 8
E = 320000
D = 128
H = 16

NC = 2
NS = 16
NW = NC * NS

EB = 128
ETB = 80
NBLK = E // EB
PADB = NW * ETB - NBLK

NODES_PER_TILE = NP // NS
WCHUNKS = NODES_PER_TILE // EB

LOOK = 4
SLOTS = 8
NGRP = ETB // SLOTS

_mesh = plsc.VectorSubcoreMesh(core_axis_name="c", subcore_axis_name="s")
_sc_params = pltpu.CompilerParams(use_tc_tiling_on_sc=False)


def _zero_rows(buf, n):
    z = jnp.zeros((H,), jnp.float32)
    for i in range(n):
        buf[i] = z


def _load_indices(ei3_hbm, pad3_hbm, eiv, w):
    @pl.when(w < NW - 1)
    def _():
        pltpu.sync_copy(ei3_hbm.at[pl.ds(w * ETB, ETB)], eiv)

    @pl.when(w == NW - 1)
    def _():
        real = NBLK - (NW - 1) * ETB
        pltpu.sync_copy(ei3_hbm.at[pl.ds((NW - 1) * ETB, real)],
                        eiv.at[pl.ds(0, real)])
        pltpu.sync_copy(pad3_hbm, eiv.at[pl.ds(real, PADB)])


def _deg_body(ei3_hbm, pad3_hbm, out_hbm, eiv, ones, zb, pk, acc, sem):
    c = lax.axis_index("c")
    s = lax.axis_index("s")
    w = c * NS + s
    _load_indices(ei3_hbm, pad3_hbm, eiv, w)
    o16 = jnp.ones((H,), jnp.float32)
    for i in range(EB // H):
        ones[pl.ds(i * H, H)] = o16
    z16 = jnp.zeros((H,), jnp.float32)
    for i in range(NODES_PER_TILE // H):
        zb[pl.ds(i * H, H)] = z16
    pltpu.sync_copy(zb, acc.at[pl.ds(s * NODES_PER_TILE, NODES_PER_TILE)])
    plsc.subcore_barrier()

    def blk(j, carry):
        pltpu.async_copy(ones, acc.at[eiv.at[j, 1]], sem, add=True)
        return carry

    lax.fori_loop(0, ETB, blk, 0)

    def drain(j, carry):
        pltpu.make_async_copy(ones, acc.at[eiv.at[j, 1]], sem).wait()
        return carry

    lax.fori_loop(0, ETB, drain, 0)
    plsc.subcore_barrier()

    pltpu.sync_copy(acc.at[pl.ds(s * NODES_PER_TILE, NODES_PER_TILE)], zb)

    def bcast(g, carry):
        v16 = zb[pl.ds(g * H, H)]
        for q in range(H):
            off = g * 256 + (q // 8) * 128 + (q % 8) * H
            pk[pl.ds(off, H)] = jnp.full((H,), v16[q], jnp.float32)
        return carry

    lax.fori_loop(0, NODES_PER_TILE // H, bcast, 0)
    flat = (c * NPP + s * (NODES_PER_TILE // 8)) * 128
    pltpu.sync_copy(pk, out_hbm.at[pl.ds(flat, NODES_PER_TILE * H)])


_deg_kernel = pl.kernel(
    _deg_body,
    out_type=jax.ShapeDtypeStruct((NC * NPP * 128,), jnp.float32),
    mesh=_mesh,
    compiler_params=_sc_params,
    scratch_types=[
        pltpu.VMEM((ETB, 2, EB), jnp.int32),
        pltpu.VMEM((EB,), jnp.float32),
        pltpu.VMEM((NODES_PER_TILE,), jnp.float32),
        pltpu.VMEM((NODES_PER_TILE * H,), jnp.float32),
        pltpu.VMEM_SHARED((NP,), jnp.float32),
        pltpu.SemaphoreType.DMA,
    ],
)


def _edge_body(hs_hbm, ei3_hbm, pad3_hbm, out_hbm, eiv, rows, tmp, acc,
               gsem, ssem):
    c = lax.axis_index("c")
    s = lax.axis_index("s")
    w = c * NS + s
    _load_indices(ei3_hbm, pad3_hbm, eiv, w)
    _zero_rows(tmp, EB)
    for k in range(WCHUNKS):
        pltpu.sync_copy(tmp, acc.at[pl.ds(s * NODES_PER_TILE + k * EB, EB)])
    plsc.subcore_barrier()

    def block(j, b, wait_prev, fire):
        pltpu.make_async_copy(
            hs_hbm.at[eiv.at[j, 0]], rows.at[b], gsem.at[b]).wait()
        pltpu.async_copy(rows.at[b], acc.at[eiv.at[j, 1]], ssem.at[b],
                         add=True)
        if fire:
            nb = (b + LOOK) % SLOTS
            if wait_prev:
                pltpu.make_async_copy(
                    rows.at[nb], acc.at[eiv.at[j, 1]], ssem.at[nb]).wait()
            pltpu.async_copy(hs_hbm.at[eiv.at[j + LOOK, 0]], rows.at[nb],
                             gsem.at[nb])

    for b in range(LOOK):
        pltpu.async_copy(hs_hbm.at[eiv.at[b, 0]], rows.at[b], gsem.at[b])
    for b in range(SLOTS):
        block(b, b, wait_prev=(b >= LOOK), fire=True)

    def grp(g, carry):
        for b in range(SLOTS):
            block(g * SLOTS + b, b, wait_prev=True, fire=True)
        return carry

    lax.fori_loop(1, NGRP - 1, grp, 0)

    for b in range(SLOTS):
        j = (NGRP - 1) * SLOTS + b
        block(j, b, wait_prev=True, fire=(b < LOOK))
    for b in range(SLOTS):
        pltpu.make_async_copy(
            rows.at[b], acc.at[eiv.at[0, 1]], ssem.at[b]).wait()

    plsc.subcore_barrier()
    for k in range(WCHUNKS):
        off = s * NODES_PER_TILE + k * EB
        pltpu.sync_copy(acc.at[pl.ds(off, EB)], tmp)
        pltpu.sync_copy(tmp, out_hbm.at[pl.ds(c * NP + off, EB)])


_edge_kernel = pl.kernel(
    _edge_body,
    out_type=jax.ShapeDtypeStruct((NC * NP, H), jnp.float32),
    mesh=_mesh,
    compiler_params=_sc_params,
    scratch_types=[
        pltpu.VMEM((ETB, 2, EB), jnp.int32),
        pltpu.VMEM((SLOTS, EB, H), jnp.float32),
        pltpu.VMEM((EB, H), jnp.float32),
        pltpu.VMEM_SHARED((NP, H), jnp.float32),
        pltpu.SemaphoreType.DMA((SLOTS,)),
        pltpu.SemaphoreType.DMA((SLOTS,)),
    ],
)


def _dinv_packed(degp_ref):
    deg = degp_ref[:NPP] + degp_ref[NPP:] + 1.0
    return lax.rsqrt(deg)


def _tc1a_body(xp8_ref, w1b_ref, h1_ref):
    h1_ref[...] = jnp.dot(xp8_ref[...], w1b_ref[...],
                          preferred_element_type=jnp.float32)


def _tc1b_body(h1_ref, degp_ref, hs1_ref):
    hs1_ref[...] = h1_ref[...] * _dinv_packed(degp_ref)


def _tc2_body(p1_ref, hs1_ref, degp_ref, b1_ref, w2b_ref, hs2_ref):
    dinv = _dinv_packed(degp_ref)
    y1 = (p1_ref[:NPP] + p1_ref[NPP:] + hs1_ref[...]) * dinv + b1_ref[...]
    y1 = jnp.maximum(y1, 0.0)
    h2p = jnp.dot(y1, w2b_ref[...], preferred_element_type=jnp.float32)
    hs2_ref[...] = h2p * dinv


def _tc3_body(p2_ref, hs2_ref, degp_ref, b2_ref, bsum_ref, out_ref):
    dinv = _dinv_packed(degp_ref)
    zp = (p2_ref[:NPP] + p2_ref[NPP:] + hs2_ref[...]) * dinv + b2_ref[...]
    m = jnp.max(zp, axis=1, keepdims=True)
    ez = jnp.exp(zp - m)
    s = jnp.dot(ez, bsum_ref[...], preferred_element_type=jnp.float32)
    out_ref[...] = (zp - m - jnp.log(s))[:N // 8]


_tc1a = pl.pallas_call(
    _tc1a_body, out_shape=jax.ShapeDtypeStruct((NPP, 128), jnp.float32))
_tc1b = pl.pallas_call(
    _tc1b_body, out_shape=jax.ShapeDtypeStruct((NPP, 128), jnp.float32))
_tc2 = pl.pallas_call(
    _tc2_body, out_shape=jax.ShapeDtypeStruct((NPP, 128), jnp.float32))
_tc3 = pl.pallas_call(
    _tc3_body, out_shape=jax.ShapeDtypeStruct((N // 8, 128), jnp.float32))


@jax.jit
def kernel(x, edge_index, W1, b1, W2, b2):
    ei3 = edge_index.astype(jnp.int32).reshape(2, NBLK, EB).transpose(1, 0, 2)
    pad3 = (N + jnp.arange(PADB * 2 * EB, dtype=jnp.int32) % (NP - N)
            ).reshape(PADB, 2, EB)
    xp8 = jnp.pad(x, ((0, NP - N), (0, 0))).reshape(NPP, 8 * D)
    eye8 = jnp.eye(8, dtype=jnp.float32)
    w1b = jnp.kron(eye8, W1)
    w2b = jnp.kron(eye8, W2)
    bsum = jnp.kron(eye8, jnp.ones((H, H), jnp.float32))
    b1p = jnp.tile(b1, 8).reshape(1, 128)
    b2p = jnp.tile(b2, 8).reshape(1, 128)

    degp = _deg_kernel(ei3, pad3).reshape(2 * NPP, 128)
    h1p = _tc1a(xp8, w1b)
    hs1p = _tc1b(h1p, degp)
    p1 = _edge_kernel(hs1p.reshape(NP, H), ei3, pad3).reshape(2 * NPP, 128)
    hs2p = _tc2(p1, hs1p, degp, b1p, w2b)
    p2 = _edge_kernel(hs2p.reshape(NP, H), ei3, pad3).reshape(2 * NPP, 128)
    outp = _tc3(p2, hs2p, degp, b2p, bsum)
    return outp.reshape(N, H)
